# baseline (speedup 1.0000x reference)
import functools

import jax
import jax.numpy as jnp
from jax import lax
from jax.experimental import pallas as pl
from jax.experimental.pallas import tpu as pltpu
from jax.experimental.pallas import tpu_sc as plsc

N = 10000
E = 320000
D = 128
BE = 2000
BN = 2000

NC = 2
NS = 16
NW = NC * NS
EW = E // NW
CK = 80
NCH = EW // CK
NR = N // NS
NZ = N // 10

_sc_mesh = plsc.VectorSubcoreMesh(core_axis_name="c", subcore_axis_name="s")
_sc_params = pltpu.CompilerParams(use_tc_tiling_on_sc=False, needs_layout_passes=False)


@functools.partial(
    pl.kernel,
    out_type=(
        jax.ShapeDtypeStruct((E, D), jnp.float32),
        jax.ShapeDtypeStruct((E, D), jnp.float32),
        jax.ShapeDtypeStruct((E,), jnp.float32),
        jax.ShapeDtypeStruct((E,), jnp.float32),
        jax.ShapeDtypeStruct((E,), jnp.float32),
    ),
    mesh=_sc_mesh,
    scratch_types=[
        pltpu.VMEM((CK,), jnp.int32),
        pltpu.VMEM((CK,), jnp.int32),
        pltpu.VMEM((CK, D), jnp.float32),
        pltpu.VMEM((CK, D), jnp.float32),
        pltpu.VMEM((N,), jnp.float32),
        pltpu.VMEM((N,), jnp.float32),
        pltpu.VMEM((N,), jnp.float32),
        pltpu.VMEM((CK,), jnp.float32),
        pltpu.VMEM((CK,), jnp.float32),
        pltpu.VMEM((CK,), jnp.float32),
        pltpu.SemaphoreType.DMA,
        pltpu.SemaphoreType.DMA,
        pltpu.SemaphoreType.DMA,
    ],
    compiler_params=_sc_params,
)
def _sc_gather(x_hbm, px_hbm, py_hbm, pz_hbm, ei_hbm, ej_hbm,
               xi_hbm, xj_hbm, dx_hbm, dy_hbm, dz_hbm,
               idxi, idxj, bi, bj, pxv, pyv, pzv, dbx, dby, dbz,
               sem0, sem1, sem2):
    pltpu.sync_copy(px_hbm, pxv)
    pltpu.sync_copy(py_hbm, pyv)
    pltpu.sync_copy(pz_hbm, pzv)
    wid = lax.axis_index("s") * NC + lax.axis_index("c")

    def body(i, carry):
        base = wid * EW + i * CK
        pltpu.sync_copy(ei_hbm.at[pl.ds(base, CK)], idxi)
        pltpu.sync_copy(ej_hbm.at[pl.ds(base, CK)], idxj)
        cpi = pltpu.async_copy(x_hbm.at[idxi], bi, sem0)
        cpj = pltpu.async_copy(x_hbm.at[idxj], bj, sem1)
        for k in range(CK // 16):
            sl = pl.ds(k * 16, 16)
            ii = idxi[sl]
            ij = idxj[sl]
            dbx[sl] = plsc.load_gather(pxv, [ij]) - plsc.load_gather(pxv, [ii])
            dby[sl] = plsc.load_gather(pyv, [ij]) - plsc.load_gather(pyv, [ii])
            dbz[sl] = plsc.load_gather(pzv, [ij]) - plsc.load_gather(pzv, [ii])
        cpi.wait()
        cpj.wait()
        pltpu.sync_copy(bi, xi_hbm.at[pl.ds(base, CK)])
        pltpu.sync_copy(bj, xj_hbm.at[pl.ds(base, CK)])
        w0 = pltpu.async_copy(dbx, dx_hbm.at[pl.ds(base, CK)], sem0)
        w1 = pltpu.async_copy(dby, dy_hbm.at[pl.ds(base, CK)], sem1)
        w2 = pltpu.async_copy(dbz, dz_hbm.at[pl.ds(base, CK)], sem2)
        w0.wait()
        w1.wait()
        w2.wait()
        return carry

    lax.fori_loop(0, NCH, body, 0)


def _edge_mlp_body(xi, xj, dx, dy, dz, w1a, w1b, w1d, b1, w2, b2,
                   msg_out, pw_out):
    dxv = dx[...]
    dyv = dy[...]
    dzv = dz[...]
    dist = jnp.sqrt(dxv * dxv + dyv * dyv + dzv * dzv + 1e-12)
    h = jnp.dot(xi[...], w1a[...], preferred_element_type=jnp.float32)
    h += jnp.dot(xj[...], w1b[...], preferred_element_type=jnp.float32)
    h += dist * w1d[...] + b1[...]
    h = jnp.maximum(h, 0.0)
    msg = jnp.dot(h, w2[...], preferred_element_type=jnp.float32) + b2[...]
    msg_out[...] = msg
    pw_out[...] = jnp.tanh(jnp.sqrt(jnp.sum(msg * msg, axis=1, keepdims=True) + 1e-12))


def _edge_mlp(xi, xj, dx, dy, dz, w1a, w1b, w1d, b1, w2, b2):
    full = lambda s: pl.BlockSpec(s, lambda i: (0, 0))
    col = pl.BlockSpec((BE, 1), lambda i: (i, 0))
    return pl.pallas_call(
        _edge_mlp_body,
        grid=(E // BE,),
        in_specs=[
            pl.BlockSpec((BE, D), lambda i: (i, 0)),
            pl.BlockSpec((BE, D), lambda i: (i, 0)),
            col, col, col,
            full((D, 2 * D)),
            full((D, 2 * D)),
            full((1, 2 * D)),
            full((1, 2 * D)),
            full((2 * D, D)),
            full((1, D)),
        ],
        out_specs=[pl.BlockSpec((BE, D), lambda i: (i, 0)), col],
        out_shape=[
            jax.ShapeDtypeStruct((E, D), jnp.float32),
            jax.ShapeDtypeStruct((E, 1), jnp.float32),
        ],
    )(xi, xj, dx.reshape(E, 1), dy.reshape(E, 1), dz.reshape(E, 1),
      w1a, w1b, w1d, b1, w2, b2)


@functools.partial(
    pl.kernel,
    out_type=(
        jax.ShapeDtypeStruct((2 * N, D), jnp.float32),
        jax.ShapeDtypeStruct((2 * N,), jnp.float32),
        jax.ShapeDtypeStruct((2 * N,), jnp.float32),
        jax.ShapeDtypeStruct((2 * N,), jnp.float32),
        jax.ShapeDtypeStruct((2 * N,), jnp.float32),
    ),
    mesh=_sc_mesh,
    scratch_types=[
        pltpu.VMEM((CK,), jnp.int32),
        pltpu.VMEM((CK, D), jnp.float32),
        pltpu.VMEM((CK,), jnp.float32),
        pltpu.VMEM((CK,), jnp.float32),
        pltpu.VMEM((CK,), jnp.float32),
        pltpu.VMEM((CK,), jnp.float32),
        pltpu.VMEM((NZ,), jnp.float32),
        pltpu.VMEM_SHARED((N, D), jnp.float32),
        pltpu.VMEM_SHARED((N,), jnp.float32),
        pltpu.VMEM_SHARED((N,), jnp.float32),
        pltpu.VMEM_SHARED((N,), jnp.float32),
        pltpu.VMEM_SHARED((N,), jnp.float32),
        pltpu.SemaphoreType.DMA,
        pltpu.SemaphoreType.DMA,
        pltpu.SemaphoreType.DMA,
        pltpu.SemaphoreType.DMA,
    ],
    compiler_params=_sc_params,
)
def _sc_scatter(msg_hbm, ex_hbm, ey_hbm, ez_hbm, ei_hbm, zm_hbm,
                aggm_hbm, sx_hbm, sy_hbm, sz_hbm, cnt_hbm,
                idx, bm, bx, by, bz, ones, zb,
                accm, accx, accy, accz, accc,
                sem0, sem1, sem2, sem3):
    c = lax.axis_index("c")
    s = lax.axis_index("s")
    wid = s * NC + c
    rows = pl.ds(s * NR, NR)
    pltpu.sync_copy(zm_hbm.at[rows], accm.at[rows])
    for k in range(NZ // 16):
        zb[pl.ds(k * 16, 16)] = jnp.zeros((16,), jnp.float32)
    for k in range(CK // 16):
        ones[pl.ds(k * 16, 16)] = jnp.ones((16,), jnp.float32)

    @pl.when(s < 10)
    def _():
        zrows = pl.ds(s * NZ, NZ)
        pltpu.sync_copy(zb, accx.at[zrows])
        pltpu.sync_copy(zb, accy.at[zrows])
        pltpu.sync_copy(zb, accz.at[zrows])
        pltpu.sync_copy(zb, accc.at[zrows])

    plsc.subcore_barrier()

    def body(i, carry):
        base = pl.ds(wid * EW + i * CK, CK)
        pltpu.sync_copy(ei_hbm.at[base], idx)
        cm = pltpu.async_copy(msg_hbm.at[base], bm, sem0)
        cx = pltpu.async_copy(ex_hbm.at[base], bx, sem1)
        cy = pltpu.async_copy(ey_hbm.at[base], by, sem2)
        cz = pltpu.async_copy(ez_hbm.at[base], bz, sem3)
        cm.wait()
        cx.wait()
        cy.wait()
        cz.wait()
        pltpu.sync_copy(bm, accm.at[idx], add=True)
        pltpu.sync_copy(bx, accx.at[idx], add=True)
        pltpu.sync_copy(by, accy.at[idx], add=True)
        pltpu.sync_copy(bz, accz.at[idx], add=True)
        pltpu.sync_copy(ones, accc.at[idx], add=True)
        return carry

    lax.fori_loop(0, NCH, body, 0)
    plsc.subcore_barrier()
    pltpu.sync_copy(accm.at[rows], aggm_hbm.at[pl.ds(c * N + s * NR, NR)])

    @pl.when(s < 10)
    def _():
        zrows = pl.ds(s * NZ, NZ)
        orows = pl.ds(c * N + s * NZ, NZ)
        pltpu.sync_copy(accx.at[zrows], sx_hbm.at[orows])
        pltpu.sync_copy(accy.at[zrows], sy_hbm.at[orows])
        pltpu.sync_copy(accz.at[zrows], sz_hbm.at[orows])
        pltpu.sync_copy(accc.at[zrows], cnt_hbm.at[orows])


def _node_update_body(x, pos, m, px_, py_, pz_, pc_,
                      w3a, w3b, b3, w4, b4, xout, posout):
    u = jnp.dot(x[...], w3a[...], preferred_element_type=jnp.float32)
    u += jnp.dot(m[...], w3b[...], preferred_element_type=jnp.float32)
    u += b3[...]
    u = jnp.maximum(u, 0.0)
    u = jnp.dot(u, w4[...], preferred_element_type=jnp.float32) + b4[...]
    z = x[...] + u
    xout[...] = jnp.where(z > 0, z, jnp.exp(jnp.minimum(z, 0.0)) - 1.0)
    cnt = jnp.maximum(pc_[...], 1.0)
    psum = jnp.concatenate([px_[...], py_[...], pz_[...]], axis=1)
    posout[...] = pos[...] + psum / cnt


def _node_update(x, pos, agg, sx, sy, sz, cnt, w3a, w3b, b3, w4, b4):
    full = lambda s: pl.BlockSpec(s, lambda i: (0, 0))
    nb = N // BN
    col = pl.BlockSpec((BN, 1), lambda i: (i, 0))
    return pl.pallas_call(
        _node_update_body,
        grid=(nb,),
        in_specs=[
            pl.BlockSpec((BN, D), lambda i: (i, 0)),
            pl.BlockSpec((BN, 3), lambda i: (i, 0)),
            pl.BlockSpec((BN, D), lambda i: (i, 0)),
            col, col, col, col,
            full((D, 2 * D)),
            full((D, 2 * D)),
            full((1, 2 * D)),
            full((2 * D, D)),
            full((1, D)),
        ],
        out_specs=[
            pl.BlockSpec((BN, D), lambda i: (i, 0)),
            pl.BlockSpec((BN, 3), lambda i: (i, 0)),
        ],
        out_shape=[
            jax.ShapeDtypeStruct((N, D), jnp.float32),
            jax.ShapeDtypeStruct((N, 3), jnp.float32),
        ],
    )(x, pos, agg, sx, sy, sz, cnt, w3a, w3b, b3, w4, b4)


def kernel(x, pos, edge_index, W1, b1, W2, b2, W3, b3, W4, b4):
    ei = edge_index[0]
    ej = edge_index[1]
    pos_ij = pos[ej] - pos[ei]
    msg, pw = _edge_mlp(
        x[ei], x[ej], pos_ij[:, 0], pos_ij[:, 1], pos_ij[:, 2],
        W1[:D], W1[D:2 * D], W1[2 * D:], b1[None, :], W2, b2[None, :],
    )
    epu = pw * pos_ij
    msg_l = (msg + 1.0) - 1.0
    zm = jnp.zeros((N, D), jnp.float32)
    agg2n, sx2, sy2, sz2, cnt2 = _sc_scatter(
        msg_l, epu[:, 0], epu[:, 1], epu[:, 2], ei, zm)
    agg = agg2n[:N] + agg2n[N:]
    comb = lambda v: (v[:N] + v[N:]).reshape(N, 1)
    x_out, pos_out = _node_update(
        x, pos, agg, comb(sx2), comb(sy2), comb(sz2), comb(cnt2),
        W3[:D], W3[D:], b3[None, :], W4, b4[None, :],
    )
    return (x_out, pos_out)
